# Initial kernel scaffold; baseline (speedup 1.0000x reference)
import jax
import jax.numpy as jnp
from jax.experimental import pallas as pl


def kernel(x, edge_index, batch, W1, b1, W2, b2, Wlin, blin):
    raise NotImplementedError("write your pallas kernel here")


# speedup vs baseline: 13.4645x; 13.4645x over previous
import functools

import jax
import jax.numpy as jnp
from jax import lax
from jax.experimental import pallas as pl
from jax.experimental.pallas import tpu as pltpu
from jax.experimental.pallas import tpu_sc as plsc

_N = 10000
_NPAD = 10240
_E = 160000
_D = 256
_DH = 128
_G = 64
_NS = 16
_NW = 2 * _NS
_C = 128
_NCH = _E // _C
_CPW = _NCH // _NW
_CPT = _NCH // _NS
_ZPT = _NPAD // _NS
_R0 = 624
_R15 = _N - 15 * _R0

_mesh = lambda: plsc.VectorSubcoreMesh(core_axis_name="c", subcore_axis_name="s")
_HIGH = lax.Precision.HIGHEST


def _off(v):
    return pl.multiple_of(v, _C)


def _sc_deg(dst, zeros_, ones_):
    @functools.partial(
        pl.kernel,
        out_type=[jax.ShapeDtypeStruct((_NPAD,), jnp.float32)] * 2,
        mesh=_mesh(),
        scratch_types=[
            pltpu.VMEM_SHARED((_NPAD,), jnp.float32),
            pltpu.VMEM((_ZPT,), jnp.float32),
            pltpu.VMEM((_C,), jnp.int32),
            pltpu.VMEM((_C,), jnp.float32),
        ],
    )
    def k(dst_hbm, zeros_hbm, ones_hbm, degA_hbm, degB_hbm,
          acc, zbuf, didx, vals):
        c = lax.axis_index("c")
        s = lax.axis_index("s")
        w = c * _NS + s
        pltpu.sync_copy(zeros_hbm, zbuf)
        pltpu.sync_copy(zbuf, acc.at[pl.ds(_off(s * _ZPT), _ZPT)])
        pltpu.sync_copy(ones_hbm, vals)
        plsc.subcore_barrier()

        def chunk(i):
            pltpu.sync_copy(dst_hbm.at[pl.ds(_off(i * _C), _C)], didx)
            pltpu.sync_copy(vals, acc.at[didx], add=True)

        @pl.loop(0, _CPW)
        def _(j):
            chunk(w * _CPW + j)

        @pl.when(w < _NCH - _NW * _CPW)
        def _():
            chunk(_NW * _CPW + w)

        plsc.subcore_barrier()

        def out(dest):
            pltpu.sync_copy(acc.at[pl.ds(_off(s * _ZPT), _ZPT)], zbuf)
            pltpu.sync_copy(zbuf, dest.at[pl.ds(_off(s * _ZPT), _ZPT)])

        @pl.when(c == 0)
        def _():
            out(degA_hbm)

        @pl.when(c == 1)
        def _():
            out(degB_hbm)

    return k(dst, zeros_, ones_)


def _sc_agg(uA, uB, src, dst):
    @functools.partial(
        pl.kernel,
        out_type=[jax.ShapeDtypeStruct((_N, _DH), jnp.float32)] * 2,
        mesh=_mesh(),
        scratch_types=[
            pltpu.VMEM_SHARED((_N, _DH), jnp.float32),
            pltpu.VMEM((_C, _DH), jnp.float32),
            pltpu.VMEM((_C,), jnp.int32),
            pltpu.VMEM((_C,), jnp.int32),
        ],
    )
    def k(uA_hbm, uB_hbm, src_hbm, dst_hbm, s1A_hbm, s1B_hbm,
          acc, rows, sidx, didx):
        c = lax.axis_index("c")
        s = lax.axis_index("s")

        def xfer_rows(src_big, dst_big):
            @pl.loop(0, 5)
            def _(j):
                i = s * 5 + j

                @pl.when(i < (_N // _C))
                def _():
                    o = pl.multiple_of(i * _C, 8)
                    pltpu.sync_copy(src_big.at[pl.ds(o, _C)], rows)
                    pltpu.sync_copy(rows, dst_big.at[pl.ds(o, _C)])

            @pl.when(s == _NS - 1)
            def _():
                o = (_N // _C) * _C
                t = _N - o
                pltpu.sync_copy(src_big.at[pl.ds(o, t)], rows.at[pl.ds(0, t)])
                pltpu.sync_copy(rows.at[pl.ds(0, t)], dst_big.at[pl.ds(o, t)])

        def run(u_hbm, out_hbm):
            xfer_rows(u_hbm, acc)
            plsc.subcore_barrier()

            def chunk(i):
                o = _off(i * _C)
                pltpu.sync_copy(src_hbm.at[pl.ds(o, _C)], sidx)
                pltpu.sync_copy(dst_hbm.at[pl.ds(o, _C)], didx)
                pltpu.sync_copy(u_hbm.at[sidx], rows)
                pltpu.sync_copy(rows, acc.at[didx], add=True)

            @pl.loop(0, _CPT)
            def _(j):
                chunk(s * _CPT + j)

            @pl.when(s < _NCH - _NS * _CPT)
            def _():
                chunk(_NS * _CPT + s)

            plsc.subcore_barrier()
            xfer_rows(acc, out_hbm)

        @pl.when(c == 0)
        def _():
            run(uA_hbm, s1A_hbm)

        @pl.when(c == 1)
        def _():
            run(uB_hbm, s1B_hbm)

    return k(uA, uB, src, dst)


def _sc_agg1(zz, src, dst, zeros_):
    @functools.partial(
        pl.kernel,
        out_type=[jax.ShapeDtypeStruct((_NPAD,), jnp.float32)] * 2,
        mesh=_mesh(),
        scratch_types=[
            pltpu.VMEM_SHARED((_NPAD,), jnp.float32),
            pltpu.VMEM((_ZPT,), jnp.float32),
            pltpu.VMEM((_C,), jnp.int32),
            pltpu.VMEM((_C,), jnp.int32),
            pltpu.VMEM((_C,), jnp.float32),
        ],
    )
    def k(zz_hbm, src_hbm, dst_hbm, zeros_hbm, s2A_hbm, s2B_hbm,
          acc, zbuf, sidx, didx, vals):
        c = lax.axis_index("c")
        s = lax.axis_index("s")
        w = c * _NS + s
        pltpu.sync_copy(zeros_hbm, zbuf)
        pltpu.sync_copy(zbuf, acc.at[pl.ds(_off(s * _ZPT), _ZPT)])
        plsc.subcore_barrier()

        def chunk(i):
            o = _off(i * _C)
            pltpu.sync_copy(src_hbm.at[pl.ds(o, _C)], sidx)
            pltpu.sync_copy(dst_hbm.at[pl.ds(o, _C)], didx)
            pltpu.sync_copy(zz_hbm.at[sidx], vals)
            pltpu.sync_copy(vals, acc.at[didx], add=True)

        @pl.loop(0, _CPW)
        def _(j):
            chunk(w * _CPW + j)

        @pl.when(w < _NCH - _NW * _CPW)
        def _():
            chunk(_NW * _CPW + w)

        plsc.subcore_barrier()

        def out(dest):
            pltpu.sync_copy(acc.at[pl.ds(_off(s * _ZPT), _ZPT)], zbuf)
            pltpu.sync_copy(zbuf, dest.at[pl.ds(_off(s * _ZPT), _ZPT)])

        @pl.when(c == 0)
        def _():
            out(s2A_hbm)

        @pl.when(c == 1)
        def _():
            out(s2B_hbm)

    return k(zz, src, dst, zeros_)


_NB = 2000


def _tc1_body(dA_ref, dB_ref, x_ref, w1_ref, uA_ref, uB_ref):
    deg = dA_ref[...] + dB_ref[...] + 1.0
    dinv = lax.rsqrt(deg)
    y = x_ref[...] * dinv
    u = jnp.dot(y, w1_ref[...], precision=_HIGH,
                preferred_element_type=jnp.float32)
    uA_ref[...] = u[:, :_DH]
    uB_ref[...] = u[:, _DH:]


def _tc1(dAc, dBc, x, W1):
    col = pl.BlockSpec((_NB, 1), lambda i: (i, 0))
    return pl.pallas_call(
        _tc1_body,
        grid=(_N // _NB,),
        in_specs=[col, col,
                  pl.BlockSpec((_NB, _D), lambda i: (i, 0)),
                  pl.BlockSpec((_D, _D), lambda i: (0, 0))],
        out_specs=[pl.BlockSpec((_NB, _DH), lambda i: (i, 0))] * 2,
        out_shape=[jax.ShapeDtypeStruct((_N, _DH), jnp.float32)] * 2,
    )(dAc, dBc, x, W1)


def _tc2_body(dA_ref, dB_ref, s1A_ref, s1B_ref, b1_ref, w2_ref, wlin_ref,
              zz_ref):
    deg = dA_ref[...] + dB_ref[...] + 1.0
    dinv = lax.rsqrt(deg)
    b1 = b1_ref[...]
    hA = jnp.maximum(dinv * s1A_ref[...] + b1[None, :_DH], 0.0)
    hB = jnp.maximum(dinv * s1B_ref[...] + b1[None, _DH:], 0.0)
    w2l = jnp.dot(w2_ref[...], wlin_ref[...], precision=_HIGH,
                  preferred_element_type=jnp.float32)
    z = (jnp.dot(hA, w2l[:_DH], precision=_HIGH,
                 preferred_element_type=jnp.float32)
         + jnp.dot(hB, w2l[_DH:], precision=_HIGH,
                   preferred_element_type=jnp.float32))
    zz_ref[...] = dinv * z


def _tc2(dAc, dBc, s1A, s1B, b1, W2, Wlin):
    col = pl.BlockSpec((_NB, 1), lambda i: (i, 0))
    half = pl.BlockSpec((_NB, _DH), lambda i: (i, 0))
    return pl.pallas_call(
        _tc2_body,
        grid=(_N // _NB,),
        in_specs=[col, col, half, half,
                  pl.BlockSpec((_D,), lambda i: (0,)),
                  pl.BlockSpec((_D, _D), lambda i: (0, 0)),
                  pl.BlockSpec((_D, 1), lambda i: (0, 0))],
        out_specs=col,
        out_shape=jax.ShapeDtypeStruct((_N, 1), jnp.float32),
    )(dAc, dBc, s1A, s1B, b1, W2, Wlin)


def _tc3_body(dA_ref, dB_ref, s2A_ref, s2B_ref, zz_ref, batch_ref, b2_ref,
              wlin_ref, blin_ref, out_ref, pacc, cacc):
    i = pl.program_id(0)
    deg = dA_ref[...] + dB_ref[...] + 1.0
    dinv = lax.rsqrt(deg)
    a2 = dinv * (s2A_ref[...] + s2B_ref[...] + zz_ref[...])
    gids = lax.broadcasted_iota(jnp.int32, (_NB, _G), 1)
    onehot = (batch_ref[...] == gids).astype(jnp.float32)

    @pl.when(i == 0)
    def _():
        pacc[...] = jnp.zeros((1, _G), jnp.float32)
        cacc[...] = jnp.zeros((1, _G), jnp.float32)

    pacc[...] += jnp.sum(a2 * onehot, axis=0, keepdims=True)
    cacc[...] += jnp.sum(onehot, axis=0, keepdims=True)

    @pl.when(i == _N // _NB - 1)
    def _():
        c2 = jnp.dot(b2_ref[...][None, :], wlin_ref[...], precision=_HIGH,
                     preferred_element_type=jnp.float32)
        counts = cacc[...]
        out_ref[...] = (pacc[...] / jnp.maximum(counts, 1.0)
                        + jnp.minimum(counts, 1.0) * c2
                        + blin_ref[...][None, :])


def _tc3(dAc, dBc, s2Ac, s2Bc, zzcol, batchcol, b2, Wlin, blin):
    col = pl.BlockSpec((_NB, 1), lambda i: (i, 0))
    return pl.pallas_call(
        _tc3_body,
        grid=(_N // _NB,),
        in_specs=[col, col, col, col, col, col,
                  pl.BlockSpec((_D,), lambda i: (0,)),
                  pl.BlockSpec((_D, 1), lambda i: (0, 0)),
                  pl.BlockSpec((1,), lambda i: (0,))],
        out_specs=pl.BlockSpec((1, _G), lambda i: (0, 0)),
        out_shape=jax.ShapeDtypeStruct((1, _G), jnp.float32),
        scratch_shapes=[pltpu.VMEM((1, _G), jnp.float32),
                        pltpu.VMEM((1, _G), jnp.float32)],
    )(dAc, dBc, s2Ac, s2Bc, zzcol, batchcol, b2, Wlin, blin)


def kernel(x, edge_index, batch, W1, b1, W2, b2, Wlin, blin):
    src = edge_index[0]
    dst = edge_index[1]
    zeros_ = jnp.zeros((_ZPT,), jnp.float32)
    ones_ = jnp.ones((_C,), jnp.float32)

    degA, degB = _sc_deg(dst, zeros_, ones_)
    dAc = jnp.reshape(degA, (_NPAD, 1))[:_N]
    dBc = jnp.reshape(degB, (_NPAD, 1))[:_N]

    uA, uB = _tc1(dAc, dBc, x, W1)
    s1A, s1B = _sc_agg(uA, uB, src, dst)
    zzcol = _tc2(dAc, dBc, s1A, s1B, b1, W2, Wlin)

    zzflat = jnp.reshape(zzcol, (_N,))
    s2A, s2B = _sc_agg1(zzflat, src, dst, zeros_)

    out = _tc3(dAc, dBc, jnp.reshape(s2A, (_NPAD, 1))[:_N],
               jnp.reshape(s2B, (_NPAD, 1))[:_N], zzcol,
               batch[:, None], b2, Wlin, blin)
    return jnp.reshape(out, (_G, 1))
